# baseline (speedup 1.0000x reference)
import functools

import jax
import jax.numpy as jnp
from jax import lax
from jax.experimental import pallas as pl
from jax.experimental.pallas import tpu as pltpu
from jax.experimental.pallas import tpu_sc as plsc

B = 8192
MAX_N = 64
NUM_ATOM_TYPES = 16
NUM_EDGE_TYPES = 5
VAL_BINS = 3 * MAX_N - 2

NC = 2
NS = 16
L = 16
NW = NC * NS
MPW = B // NW
GRP = 2

ROW_N = 0
ROW_NODE = ROW_N + MAX_N + 1
ROW_EDGE = ROW_NODE + NUM_ATOM_TYPES
ROW_VAL = ROW_EDGE + NUM_EDGE_TYPES
HIST_ROWS = 288
HIST_WORDS = HIST_ROWS * L


def _sc_body(e_hbm, a_hbm, n_hbm, out_hbm, ebuf0, ebuf1, abuf, nbuf, hist,
             sem0, sem1):
    cid = lax.axis_index("c")
    sid = lax.axis_index("s")
    wid = sid * NC + cid
    base = wid * MPW

    zeros = jnp.zeros((L,), jnp.float32)
    ones = jnp.ones((L,), jnp.float32)
    lane = lax.iota(jnp.int32, L)

    def zero_row(k, _):
        hist[pl.ds(k * L, L)] = zeros
        return 0
    lax.fori_loop(0, HIST_ROWS, zero_row, 0)

    pltpu.sync_copy(n_hbm.at[pl.ds(base, MPW)], nbuf)
    pltpu.sync_copy(a_hbm.at[pl.ds(base * MAX_N, MPW * MAX_N)], abuf)

    true_mask = lane < L

    def n_hist(k, _):
        nv = nbuf[pl.ds(k * L, L)]
        idx = (ROW_N + nv) * L + lane
        plsc.addupdate_scatter(hist, [idx], ones, mask=true_mask)
        return 0
    lax.fori_loop(0, MPW // L, n_hist, 0)

    jvecs = [lane + c * L for c in range(MAX_N // L)]
    ebufs = (ebuf0, ebuf1)
    sems = (sem0, sem1)

    def dma_start(g, k):
        start = jnp.minimum(base + g * GRP, B - GRP) * (MAX_N * MAX_N)
        src = e_hbm.at[pl.ds(start, GRP * MAX_N * MAX_N)]
        pltpu.async_copy(src, ebufs[k], sems[k])

    def dma_wait(k):
        src = e_hbm.at[pl.ds(0, GRP * MAX_N * MAX_N)]
        pltpu.make_async_copy(src, ebufs[k], sems[k]).wait()

    zero4 = tuple(jnp.zeros((L,), jnp.float32) for _ in range(4))

    def process(m, ebuf, j, moms):
        nv = plsc.load_gather(nbuf, [jnp.full((L,), m, jnp.int32)])
        n_s = jnp.max(nv)
        colmask = [jv < nv for jv in jvecs]
        ebase = j * (MAX_N * MAX_N)

        def row(i, carry):
            accs, (s1, s2, s3, s4) = carry
            iv = jnp.full((L,), i, jnp.int32)
            new_accs = []
            for c in range(MAX_N // L):
                et = ebuf[pl.ds(ebase + i * MAX_N + c * L, L)]
                etf = et.astype(jnp.float32)
                vf = jnp.where(et == 4, jnp.float32(1.5), etf)
                new_accs.append(accs[c] + vf)
                m1 = colmask[c] & (jvecs[c] > iv)
                em = jnp.where(m1, etf, jnp.float32(0.0))
                e2 = em * em
                s1 = s1 + em
                s2 = s2 + e2
                s3 = s3 + e2 * em
                s4 = s4 + e2 * e2
            return tuple(new_accs), (s1, s2, s3, s4)

        accs, moms = lax.fori_loop(0, n_s, row, (zero4, moms))

        for c in range(MAX_N // L):
            vb = jnp.clip(accs[c].astype(jnp.int32), 0, VAL_BINS - 1)
            vidx = (ROW_VAL + vb) * L + lane
            plsc.addupdate_scatter(hist, [vidx], ones, mask=colmask[c])
            at = abuf[pl.ds(m * MAX_N + c * L, L)]
            aidx = (ROW_NODE + at) * L + lane
            plsc.addupdate_scatter(hist, [aidx], ones, mask=colmask[c])
        return moms

    dma_start(0, 0)

    def pair(p, moms):
        g = p * 2
        dma_wait(0)
        dma_start(g + 1, 1)
        moms = lax.fori_loop(
            0, GRP, lambda j, mm: process(g * GRP + j, ebuf0, j, mm), moms)
        dma_wait(1)
        dma_start(g + 2, 0)
        moms = lax.fori_loop(
            0, GRP, lambda j, mm: process((g + 1) * GRP + j, ebuf1, j, mm),
            moms)
        return moms

    moms = lax.fori_loop(0, MPW // (2 * GRP), pair, zero4)
    dma_wait(0)
    for k in range(4):
        hist[pl.ds((ROW_EDGE + 1 + k) * L, L)] = moms[k]
    pltpu.sync_copy(hist, out_hbm.at[wid])


def _finalize_body(p_ref, tn_ref, tnode_ref, tedge_ref, tval_ref,
                   on_ref, onode_ref, oedge_ref, oval_ref, omae_ref):
    p = p_ref[...]
    s = jnp.sum(jnp.sum(p, axis=0), axis=1)

    hn = s[ROW_N:ROW_N + MAX_N + 1]
    hnode = s[ROW_NODE:ROW_NODE + NUM_ATOM_TYPES]
    hval = s[ROW_VAL:ROW_VAL + VAL_BINS]

    iv = lax.broadcasted_iota(jnp.int32, (1, MAX_N + 1), 1).astype(jnp.float32)
    s0 = jnp.sum(hn.reshape(1, MAX_N + 1) * iv * (iv - 1.0) * 0.5)
    s1 = s[ROW_EDGE + 1]
    s2 = s[ROW_EDGE + 2]
    s3 = s[ROW_EDGE + 3]
    s4 = s[ROW_EDGE + 4]
    c1 = 4.0 * s1 - (13.0 / 3.0) * s2 + 1.5 * s3 - (1.0 / 6.0) * s4
    c2 = -3.0 * s1 + (19.0 / 4.0) * s2 - 2.0 * s3 + 0.25 * s4
    c3 = (4.0 / 3.0) * s1 - (7.0 / 3.0) * s2 + (7.0 / 6.0) * s3 \
        - (1.0 / 6.0) * s4
    c4 = -0.25 * s1 + (11.0 / 24.0) * s2 - 0.25 * s3 + (1.0 / 24.0) * s4
    c0 = s0 - c1 - c2 - c3 - c4
    hedge = jnp.concatenate(
        [c0[None], c1[None], c2[None], c3[None], c4[None]])

    gn = hn / jnp.sum(hn)
    gnode = hnode / jnp.sum(hnode)
    gedge = hedge / jnp.sum(hedge)
    gval = hval / jnp.sum(hval)

    tn = tn_ref[...]
    tn = tn / jnp.sum(tn)
    tnode = tnode_ref[...]
    tnode = tnode / jnp.sum(tnode)
    tedge = tedge_ref[...]
    tedge = tedge / jnp.sum(tedge)
    tval = tval_ref[...]
    tval = tval / jnp.sum(tval)

    on_ref[...] = gn
    onode_ref[...] = gnode
    oedge_ref[...] = gedge
    oval_ref[...] = gval
    omae_ref[...] = jnp.concatenate([
        jnp.mean(jnp.abs(gn - tn))[None],
        jnp.mean(jnp.abs(gnode - tnode))[None],
        jnp.mean(jnp.abs(gedge - tedge))[None],
        jnp.mean(jnp.abs(gval - tval))[None],
    ])


@jax.jit
def kernel(atom_types, edge_types, n_nodes, n_target_dist, node_target_dist,
           edge_target_dist, valency_target_dist):
    a2 = jnp.asarray(atom_types, jnp.int32).reshape(B * MAX_N)
    e2 = jnp.asarray(edge_types, jnp.int32).reshape(B * MAX_N * MAX_N)
    nn = jnp.asarray(n_nodes, jnp.int32)

    mesh = plsc.VectorSubcoreMesh(core_axis_name="c", subcore_axis_name="s")
    partials = pl.kernel(
        _sc_body,
        out_type=jax.ShapeDtypeStruct((NW, HIST_WORDS), jnp.float32),
        mesh=mesh,
        compiler_params=pltpu.CompilerParams(needs_layout_passes=False),
        scratch_types=[
            pltpu.VMEM((GRP * MAX_N * MAX_N,), jnp.int32),
            pltpu.VMEM((GRP * MAX_N * MAX_N,), jnp.int32),
            pltpu.VMEM((MPW * MAX_N,), jnp.int32),
            pltpu.VMEM((MPW,), jnp.int32),
            pltpu.VMEM((HIST_WORDS,), jnp.float32),
            pltpu.SemaphoreType.DMA,
            pltpu.SemaphoreType.DMA,
        ],
    )(e2, a2, nn)

    p3 = partials.reshape(NW, HIST_ROWS, L)
    return pl.pallas_call(
        _finalize_body,
        out_shape=(
            jax.ShapeDtypeStruct((MAX_N + 1,), jnp.float32),
            jax.ShapeDtypeStruct((NUM_ATOM_TYPES,), jnp.float32),
            jax.ShapeDtypeStruct((NUM_EDGE_TYPES,), jnp.float32),
            jax.ShapeDtypeStruct((VAL_BINS,), jnp.float32),
            jax.ShapeDtypeStruct((4,), jnp.float32),
        ),
    )(p3, n_target_dist, node_target_dist, edge_target_dist,
      valency_target_dist)


# speedup vs baseline: 1.0188x; 1.0188x over previous
import functools

import jax
import jax.numpy as jnp
from jax import lax
from jax.experimental import pallas as pl
from jax.experimental.pallas import tpu as pltpu
from jax.experimental.pallas import tpu_sc as plsc

B = 8192
MAX_N = 64
NUM_ATOM_TYPES = 16
NUM_EDGE_TYPES = 5
VAL_BINS = 3 * MAX_N - 2

NC = 2
NS = 16
L = 16
NW = NC * NS
MPW = B // NW
GRP = 8

ROW_N = 0
ROW_NODE = ROW_N + MAX_N + 1
ROW_EDGE = ROW_NODE + NUM_ATOM_TYPES
ROW_VAL = ROW_EDGE + NUM_EDGE_TYPES
HIST_ROWS = 288
HIST_WORDS = HIST_ROWS * L


def _sc_body(e_hbm, a_hbm, n_hbm, out_hbm, ebuf0, ebuf1, abuf, nbuf, hist,
             sem0, sem1):
    cid = lax.axis_index("c")
    sid = lax.axis_index("s")
    wid = sid * NC + cid
    base = wid * MPW

    zeros = jnp.zeros((L,), jnp.float32)
    ones = jnp.ones((L,), jnp.float32)
    lane = lax.iota(jnp.int32, L)

    def zero_row(k, _):
        hist[pl.ds(k * L, L)] = zeros
        return 0
    lax.fori_loop(0, HIST_ROWS, zero_row, 0)

    pltpu.sync_copy(n_hbm.at[pl.ds(base, MPW)], nbuf)
    pltpu.sync_copy(a_hbm.at[pl.ds(base * MAX_N, MPW * MAX_N)], abuf)

    true_mask = lane < L

    def n_hist(k, _):
        nv = nbuf[pl.ds(k * L, L)]
        idx = (ROW_N + nv) * L + lane
        plsc.addupdate_scatter(hist, [idx], ones, mask=true_mask)
        return 0
    lax.fori_loop(0, MPW // L, n_hist, 0)

    jvecs = [lane + c * L for c in range(MAX_N // L)]
    ebufs = (ebuf0, ebuf1)
    sems = (sem0, sem1)

    def dma_start(g, k):
        start = jnp.minimum(base + g * GRP, B - GRP) * (MAX_N * MAX_N)
        src = e_hbm.at[pl.ds(start, GRP * MAX_N * MAX_N)]
        pltpu.async_copy(src, ebufs[k], sems[k])

    def dma_wait(k):
        src = e_hbm.at[pl.ds(0, GRP * MAX_N * MAX_N)]
        pltpu.make_async_copy(src, ebufs[k], sems[k]).wait()

    zero4 = tuple(jnp.zeros((L,), jnp.float32) for _ in range(4))

    def process(m, ebuf, j, moms):
        nv = plsc.load_gather(nbuf, [jnp.full((L,), m, jnp.int32)])
        n_s = jnp.max(nv)
        colmask = [jv < nv for jv in jvecs]
        ebase = j * (MAX_N * MAX_N)

        def row(i, carry):
            accs, (s1, s2, s3, s4) = carry
            iv = jnp.full((L,), i, jnp.int32)
            new_accs = []
            for c in range(MAX_N // L):
                et = ebuf[pl.ds(ebase + i * MAX_N + c * L, L)]
                etf = et.astype(jnp.float32)
                vf = jnp.where(et == 4, jnp.float32(1.5), etf)
                new_accs.append(accs[c] + vf)
                m1 = colmask[c] & (jvecs[c] > iv)
                em = jnp.where(m1, etf, jnp.float32(0.0))
                e2 = em * em
                s1 = s1 + em
                s2 = s2 + e2
                s3 = s3 + e2 * em
                s4 = s4 + e2 * e2
            return tuple(new_accs), (s1, s2, s3, s4)

        accs, moms = lax.fori_loop(0, n_s, row, (zero4, moms))

        for c in range(MAX_N // L):
            vb = jnp.clip(accs[c].astype(jnp.int32), 0, VAL_BINS - 1)
            vidx = (ROW_VAL + vb) * L + lane
            plsc.addupdate_scatter(hist, [vidx], ones, mask=colmask[c])
            at = abuf[pl.ds(m * MAX_N + c * L, L)]
            aidx = (ROW_NODE + at) * L + lane
            plsc.addupdate_scatter(hist, [aidx], ones, mask=colmask[c])
        return moms

    dma_start(0, 0)

    def pair(p, moms):
        g = p * 2
        dma_wait(0)
        dma_start(g + 1, 1)
        for j in range(GRP):
            moms = process(g * GRP + j, ebuf0, j, moms)
        dma_wait(1)
        dma_start(g + 2, 0)
        for j in range(GRP):
            moms = process((g + 1) * GRP + j, ebuf1, j, moms)
        return moms

    moms = lax.fori_loop(0, MPW // (2 * GRP), pair, zero4)
    dma_wait(0)
    for k in range(4):
        hist[pl.ds((ROW_EDGE + 1 + k) * L, L)] = moms[k]
    pltpu.sync_copy(hist, out_hbm.at[wid])


def _finalize_body(p_ref, tn_ref, tnode_ref, tedge_ref, tval_ref,
                   on_ref, onode_ref, oedge_ref, oval_ref, omae_ref):
    p = p_ref[...]
    s = jnp.sum(jnp.sum(p, axis=0), axis=1)

    hn = s[ROW_N:ROW_N + MAX_N + 1]
    hnode = s[ROW_NODE:ROW_NODE + NUM_ATOM_TYPES]
    hval = s[ROW_VAL:ROW_VAL + VAL_BINS]

    iv = lax.broadcasted_iota(jnp.int32, (1, MAX_N + 1), 1).astype(jnp.float32)
    s0 = jnp.sum(hn.reshape(1, MAX_N + 1) * iv * (iv - 1.0) * 0.5)
    s1 = s[ROW_EDGE + 1]
    s2 = s[ROW_EDGE + 2]
    s3 = s[ROW_EDGE + 3]
    s4 = s[ROW_EDGE + 4]
    c1 = 4.0 * s1 - (13.0 / 3.0) * s2 + 1.5 * s3 - (1.0 / 6.0) * s4
    c2 = -3.0 * s1 + (19.0 / 4.0) * s2 - 2.0 * s3 + 0.25 * s4
    c3 = (4.0 / 3.0) * s1 - (7.0 / 3.0) * s2 + (7.0 / 6.0) * s3 \
        - (1.0 / 6.0) * s4
    c4 = -0.25 * s1 + (11.0 / 24.0) * s2 - 0.25 * s3 + (1.0 / 24.0) * s4
    c0 = s0 - c1 - c2 - c3 - c4
    hedge = jnp.concatenate(
        [c0[None], c1[None], c2[None], c3[None], c4[None]])

    gn = hn / jnp.sum(hn)
    gnode = hnode / jnp.sum(hnode)
    gedge = hedge / jnp.sum(hedge)
    gval = hval / jnp.sum(hval)

    tn = tn_ref[...]
    tn = tn / jnp.sum(tn)
    tnode = tnode_ref[...]
    tnode = tnode / jnp.sum(tnode)
    tedge = tedge_ref[...]
    tedge = tedge / jnp.sum(tedge)
    tval = tval_ref[...]
    tval = tval / jnp.sum(tval)

    on_ref[...] = gn
    onode_ref[...] = gnode
    oedge_ref[...] = gedge
    oval_ref[...] = gval
    omae_ref[...] = jnp.concatenate([
        jnp.mean(jnp.abs(gn - tn))[None],
        jnp.mean(jnp.abs(gnode - tnode))[None],
        jnp.mean(jnp.abs(gedge - tedge))[None],
        jnp.mean(jnp.abs(gval - tval))[None],
    ])


@jax.jit
def kernel(atom_types, edge_types, n_nodes, n_target_dist, node_target_dist,
           edge_target_dist, valency_target_dist):
    a2 = jnp.asarray(atom_types, jnp.int32).reshape(B * MAX_N)
    e2 = jnp.asarray(edge_types, jnp.int32).reshape(B * MAX_N * MAX_N)
    nn = jnp.asarray(n_nodes, jnp.int32)

    mesh = plsc.VectorSubcoreMesh(core_axis_name="c", subcore_axis_name="s")
    partials = pl.kernel(
        _sc_body,
        out_type=jax.ShapeDtypeStruct((NW, HIST_WORDS), jnp.float32),
        mesh=mesh,
        compiler_params=pltpu.CompilerParams(needs_layout_passes=False),
        scratch_types=[
            pltpu.VMEM((GRP * MAX_N * MAX_N,), jnp.int32),
            pltpu.VMEM((GRP * MAX_N * MAX_N,), jnp.int32),
            pltpu.VMEM((MPW * MAX_N,), jnp.int32),
            pltpu.VMEM((MPW,), jnp.int32),
            pltpu.VMEM((HIST_WORDS,), jnp.float32),
            pltpu.SemaphoreType.DMA,
            pltpu.SemaphoreType.DMA,
        ],
    )(e2, a2, nn)

    p3 = partials.reshape(NW, HIST_ROWS, L)
    return pl.pallas_call(
        _finalize_body,
        out_shape=(
            jax.ShapeDtypeStruct((MAX_N + 1,), jnp.float32),
            jax.ShapeDtypeStruct((NUM_ATOM_TYPES,), jnp.float32),
            jax.ShapeDtypeStruct((NUM_EDGE_TYPES,), jnp.float32),
            jax.ShapeDtypeStruct((VAL_BINS,), jnp.float32),
            jax.ShapeDtypeStruct((4,), jnp.float32),
        ),
    )(p3, n_target_dist, node_target_dist, edge_target_dist,
      valency_target_dist)


# speedup vs baseline: 1.6852x; 1.6540x over previous
import functools

import jax
import jax.numpy as jnp
from jax import lax
from jax.experimental import pallas as pl
from jax.experimental.pallas import tpu as pltpu
from jax.experimental.pallas import tpu_sc as plsc

B = 8192
MAX_N = 64
NUM_ATOM_TYPES = 16
NUM_EDGE_TYPES = 5
VAL_BINS = 3 * MAX_N - 2

NC = 2
NS = 16
L = 16
NW = NC * NS
MPW = B // NW
GRP = 8

ROW_N = 0
ROW_NODE = ROW_N + MAX_N + 1
ROW_EDGE = ROW_NODE + NUM_ATOM_TYPES
ROW_VAL = ROW_EDGE + NUM_EDGE_TYPES
HIST_ROWS = 288
HIST_WORDS = HIST_ROWS * L


def _sc_body(e_hbm, a_hbm, n_hbm, out_hbm, ebuf0, ebuf1, ebuf2, ebuf3,
             abuf, nbuf, hist, sem0, sem1, sem2, sem3):
    cid = lax.axis_index("c")
    sid = lax.axis_index("s")
    wid = sid * NC + cid
    base = wid * MPW

    zeros = jnp.zeros((L,), jnp.float32)
    ones = jnp.ones((L,), jnp.float32)
    lane = lax.iota(jnp.int32, L)

    def zero_row(k, _):
        hist[pl.ds(k * L, L)] = zeros
        return 0
    lax.fori_loop(0, HIST_ROWS, zero_row, 0)

    pltpu.sync_copy(n_hbm.at[pl.ds(base, MPW)], nbuf)
    pltpu.sync_copy(a_hbm.at[pl.ds(base * MAX_N, MPW * MAX_N)], abuf)

    true_mask = lane < L

    def n_hist(k, _):
        nv = nbuf[pl.ds(k * L, L)]
        idx = (ROW_N + nv) * L + lane
        plsc.addupdate_scatter(hist, [idx], ones, mask=true_mask)
        return 0
    lax.fori_loop(0, MPW // L, n_hist, 0)

    jvecs = [lane + c * L for c in range(MAX_N // L)]
    ebufs = (ebuf0, ebuf1, ebuf2, ebuf3)
    sems = (sem0, sem1, sem2, sem3)
    NBUF = 4

    def dma_start(m, k):
        src = e_hbm.at[jnp.minimum(base + m, B - 1)]
        pltpu.async_copy(src, ebufs[k], sems[k])

    def dma_wait(k):
        pltpu.make_async_copy(e_hbm.at[base], ebufs[k], sems[k]).wait()

    zero4 = tuple(jnp.zeros((L,), jnp.float32) for _ in range(4))

    def process(m, ebuf, moms):
        nv = plsc.load_gather(nbuf, [jnp.full((L,), m, jnp.int32)])
        n_s = jnp.max(nv)
        colmask = [jv < nv for jv in jvecs]
        ebase = 0

        def row(i, carry):
            accs, (s1, s2, s3, s4) = carry
            iv = jnp.full((L,), i, jnp.int32)
            new_accs = []
            for c in range(MAX_N // L):
                et = ebuf[pl.ds(ebase + i * MAX_N + c * L, L)]
                etf = et.astype(jnp.float32)
                vf = jnp.where(et == 4, jnp.float32(1.5), etf)
                new_accs.append(accs[c] + vf)
                m1 = colmask[c] & (jvecs[c] > iv)
                em = jnp.where(m1, etf, jnp.float32(0.0))
                e2 = em * em
                s1 = s1 + em
                s2 = s2 + e2
                s3 = s3 + e2 * em
                s4 = s4 + e2 * e2
            return tuple(new_accs), (s1, s2, s3, s4)

        accs, moms = lax.fori_loop(0, n_s, row, (zero4, moms))

        for c in range(MAX_N // L):
            vb = jnp.clip(accs[c].astype(jnp.int32), 0, VAL_BINS - 1)
            vidx = (ROW_VAL + vb) * L + lane
            plsc.addupdate_scatter(hist, [vidx], ones, mask=colmask[c])
            at = abuf[pl.ds(m * MAX_N + c * L, L)]
            aidx = (ROW_NODE + at) * L + lane
            plsc.addupdate_scatter(hist, [aidx], ones, mask=colmask[c])
        return moms

    for k in range(NBUF - 1):
        dma_start(k, k)

    def quad(p, moms):
        m = p * NBUF
        for k in range(NBUF):
            dma_wait(k)
            dma_start(m + k + NBUF - 1, (k + NBUF - 1) % NBUF)
            moms = process(m + k, ebufs[k], moms)
        return moms

    moms = lax.fori_loop(0, MPW // NBUF, quad, zero4)
    for k in range(NBUF - 1):
        dma_wait(k)
    for k in range(4):
        hist[pl.ds((ROW_EDGE + 1 + k) * L, L)] = moms[k]
    pltpu.sync_copy(hist, out_hbm.at[wid])


def _finalize_body(p_ref, tn_ref, tnode_ref, tedge_ref, tval_ref,
                   on_ref, onode_ref, oedge_ref, oval_ref, omae_ref):
    p = p_ref[...]
    s = jnp.sum(jnp.sum(p, axis=0), axis=1)

    hn = s[ROW_N:ROW_N + MAX_N + 1]
    hnode = s[ROW_NODE:ROW_NODE + NUM_ATOM_TYPES]
    hval = s[ROW_VAL:ROW_VAL + VAL_BINS]

    iv = lax.broadcasted_iota(jnp.int32, (1, MAX_N + 1), 1).astype(jnp.float32)
    s0 = jnp.sum(hn.reshape(1, MAX_N + 1) * iv * (iv - 1.0) * 0.5)
    s1 = s[ROW_EDGE + 1]
    s2 = s[ROW_EDGE + 2]
    s3 = s[ROW_EDGE + 3]
    s4 = s[ROW_EDGE + 4]
    c1 = 4.0 * s1 - (13.0 / 3.0) * s2 + 1.5 * s3 - (1.0 / 6.0) * s4
    c2 = -3.0 * s1 + (19.0 / 4.0) * s2 - 2.0 * s3 + 0.25 * s4
    c3 = (4.0 / 3.0) * s1 - (7.0 / 3.0) * s2 + (7.0 / 6.0) * s3 \
        - (1.0 / 6.0) * s4
    c4 = -0.25 * s1 + (11.0 / 24.0) * s2 - 0.25 * s3 + (1.0 / 24.0) * s4
    c0 = s0 - c1 - c2 - c3 - c4
    hedge = jnp.concatenate(
        [c0[None], c1[None], c2[None], c3[None], c4[None]])

    gn = hn / jnp.sum(hn)
    gnode = hnode / jnp.sum(hnode)
    gedge = hedge / jnp.sum(hedge)
    gval = hval / jnp.sum(hval)

    tn = tn_ref[...]
    tn = tn / jnp.sum(tn)
    tnode = tnode_ref[...]
    tnode = tnode / jnp.sum(tnode)
    tedge = tedge_ref[...]
    tedge = tedge / jnp.sum(tedge)
    tval = tval_ref[...]
    tval = tval / jnp.sum(tval)

    on_ref[...] = gn
    onode_ref[...] = gnode
    oedge_ref[...] = gedge
    oval_ref[...] = gval
    omae_ref[...] = jnp.concatenate([
        jnp.mean(jnp.abs(gn - tn))[None],
        jnp.mean(jnp.abs(gnode - tnode))[None],
        jnp.mean(jnp.abs(gedge - tedge))[None],
        jnp.mean(jnp.abs(gval - tval))[None],
    ])


@jax.jit
def kernel(atom_types, edge_types, n_nodes, n_target_dist, node_target_dist,
           edge_target_dist, valency_target_dist):
    a2 = jnp.asarray(atom_types, jnp.int32).reshape(B * MAX_N)
    e2 = jnp.asarray(edge_types, jnp.int32).reshape(B, MAX_N * MAX_N)
    nn = jnp.asarray(n_nodes, jnp.int32)

    mesh = plsc.VectorSubcoreMesh(core_axis_name="c", subcore_axis_name="s")
    partials = pl.kernel(
        _sc_body,
        out_type=jax.ShapeDtypeStruct((NW, HIST_WORDS), jnp.float32),
        mesh=mesh,
        compiler_params=pltpu.CompilerParams(needs_layout_passes=False),
        scratch_types=[
            pltpu.VMEM((MAX_N * MAX_N,), jnp.int32),
            pltpu.VMEM((MAX_N * MAX_N,), jnp.int32),
            pltpu.VMEM((MAX_N * MAX_N,), jnp.int32),
            pltpu.VMEM((MAX_N * MAX_N,), jnp.int32),
            pltpu.VMEM((MPW * MAX_N,), jnp.int32),
            pltpu.VMEM((MPW,), jnp.int32),
            pltpu.VMEM((HIST_WORDS,), jnp.float32),
            pltpu.SemaphoreType.DMA,
            pltpu.SemaphoreType.DMA,
            pltpu.SemaphoreType.DMA,
            pltpu.SemaphoreType.DMA,
        ],
    )(e2, a2, nn)

    p3 = partials.reshape(NW, HIST_ROWS, L)
    return pl.pallas_call(
        _finalize_body,
        out_shape=(
            jax.ShapeDtypeStruct((MAX_N + 1,), jnp.float32),
            jax.ShapeDtypeStruct((NUM_ATOM_TYPES,), jnp.float32),
            jax.ShapeDtypeStruct((NUM_EDGE_TYPES,), jnp.float32),
            jax.ShapeDtypeStruct((VAL_BINS,), jnp.float32),
            jax.ShapeDtypeStruct((4,), jnp.float32),
        ),
    )(p3, n_target_dist, node_target_dist, edge_target_dist,
      valency_target_dist)


# speedup vs baseline: 1.6897x; 1.0027x over previous
import functools

import jax
import jax.numpy as jnp
from jax import lax
from jax.experimental import pallas as pl
from jax.experimental.pallas import tpu as pltpu
from jax.experimental.pallas import tpu_sc as plsc

B = 8192
MAX_N = 64
NUM_ATOM_TYPES = 16
NUM_EDGE_TYPES = 5
VAL_BINS = 3 * MAX_N - 2

NC = 2
NS = 16
L = 16
NW = NC * NS
MPW = B // NW
GRP = 8

ROW_N = 0
ROW_NODE = ROW_N + MAX_N + 1
ROW_EDGE = ROW_NODE + NUM_ATOM_TYPES
ROW_VAL = ROW_EDGE + NUM_EDGE_TYPES
HIST_ROWS = 288
HIST_WORDS = HIST_ROWS * L


def _sc_body(e_hbm, a_hbm, n_hbm, out_hbm, ebuf0, ebuf1, ebuf2, ebuf3,
             ebuf4, ebuf5, ebuf6, ebuf7, abuf, nbuf, hist,
             sem0, sem1, sem2, sem3, sem4, sem5, sem6, sem7):
    cid = lax.axis_index("c")
    sid = lax.axis_index("s")
    wid = sid * NC + cid
    base = wid * MPW

    zeros = jnp.zeros((L,), jnp.float32)
    ones = jnp.ones((L,), jnp.float32)
    lane = lax.iota(jnp.int32, L)

    def zero_row(k, _):
        hist[pl.ds(k * L, L)] = zeros
        return 0
    lax.fori_loop(0, HIST_ROWS, zero_row, 0)

    pltpu.sync_copy(n_hbm.at[pl.ds(base, MPW)], nbuf)
    pltpu.sync_copy(a_hbm.at[pl.ds(base * MAX_N, MPW * MAX_N)], abuf)

    true_mask = lane < L

    def n_hist(k, _):
        nv = nbuf[pl.ds(k * L, L)]
        idx = (ROW_N + nv) * L + lane
        plsc.addupdate_scatter(hist, [idx], ones, mask=true_mask)
        return 0
    lax.fori_loop(0, MPW // L, n_hist, 0)

    jvecs = [lane + c * L for c in range(MAX_N // L)]
    ebufs = (ebuf0, ebuf1, ebuf2, ebuf3, ebuf4, ebuf5, ebuf6, ebuf7)
    sems = (sem0, sem1, sem2, sem3, sem4, sem5, sem6, sem7)
    NBUF = 8

    def dma_start(m, k):
        src = e_hbm.at[jnp.minimum(base + m, B - 1)]
        pltpu.async_copy(src, ebufs[k], sems[k])

    def dma_wait(k):
        pltpu.make_async_copy(e_hbm.at[base], ebufs[k], sems[k]).wait()

    zero4 = tuple(jnp.zeros((L,), jnp.float32) for _ in range(4))

    def process(m, ebuf, moms):
        nv = plsc.load_gather(nbuf, [jnp.full((L,), m, jnp.int32)])
        n_s = jnp.max(nv)
        colmask = [jv < nv for jv in jvecs]
        ebase = 0

        def row(i, carry):
            accs, (s1, s2, s3, s4) = carry
            iv = jnp.full((L,), i, jnp.int32)
            new_accs = []
            for c in range(MAX_N // L):
                et = ebuf[pl.ds(ebase + i * MAX_N + c * L, L)]
                etf = et.astype(jnp.float32)
                vf = jnp.where(et == 4, jnp.float32(1.5), etf)
                new_accs.append(accs[c] + vf)
                m1 = colmask[c] & (jvecs[c] > iv)
                em = jnp.where(m1, etf, jnp.float32(0.0))
                e2 = em * em
                s1 = s1 + em
                s2 = s2 + e2
                s3 = s3 + e2 * em
                s4 = s4 + e2 * e2
            return tuple(new_accs), (s1, s2, s3, s4)

        accs, moms = lax.fori_loop(0, n_s, row, (zero4, moms))

        for c in range(MAX_N // L):
            vb = jnp.clip(accs[c].astype(jnp.int32), 0, VAL_BINS - 1)
            vidx = (ROW_VAL + vb) * L + lane
            plsc.addupdate_scatter(hist, [vidx], ones, mask=colmask[c])
            at = abuf[pl.ds(m * MAX_N + c * L, L)]
            aidx = (ROW_NODE + at) * L + lane
            plsc.addupdate_scatter(hist, [aidx], ones, mask=colmask[c])
        return moms

    for k in range(NBUF - 1):
        dma_start(k, k)

    def quad(p, moms):
        m = p * NBUF
        for k in range(NBUF):
            dma_wait(k)
            dma_start(m + k + NBUF - 1, (k + NBUF - 1) % NBUF)
            moms = process(m + k, ebufs[k], moms)
        return moms

    moms = lax.fori_loop(0, MPW // NBUF, quad, zero4)
    for k in range(NBUF - 1):
        dma_wait(k)
    for k in range(4):
        hist[pl.ds((ROW_EDGE + 1 + k) * L, L)] = moms[k]
    pltpu.sync_copy(hist, out_hbm.at[wid])


def _finalize_body(p_ref, tn_ref, tnode_ref, tedge_ref, tval_ref,
                   on_ref, onode_ref, oedge_ref, oval_ref, omae_ref):
    p = p_ref[...]
    s = jnp.sum(jnp.sum(p, axis=0), axis=1)

    hn = s[ROW_N:ROW_N + MAX_N + 1]
    hnode = s[ROW_NODE:ROW_NODE + NUM_ATOM_TYPES]
    hval = s[ROW_VAL:ROW_VAL + VAL_BINS]

    iv = lax.broadcasted_iota(jnp.int32, (1, MAX_N + 1), 1).astype(jnp.float32)
    s0 = jnp.sum(hn.reshape(1, MAX_N + 1) * iv * (iv - 1.0) * 0.5)
    s1 = s[ROW_EDGE + 1]
    s2 = s[ROW_EDGE + 2]
    s3 = s[ROW_EDGE + 3]
    s4 = s[ROW_EDGE + 4]
    c1 = 4.0 * s1 - (13.0 / 3.0) * s2 + 1.5 * s3 - (1.0 / 6.0) * s4
    c2 = -3.0 * s1 + (19.0 / 4.0) * s2 - 2.0 * s3 + 0.25 * s4
    c3 = (4.0 / 3.0) * s1 - (7.0 / 3.0) * s2 + (7.0 / 6.0) * s3 \
        - (1.0 / 6.0) * s4
    c4 = -0.25 * s1 + (11.0 / 24.0) * s2 - 0.25 * s3 + (1.0 / 24.0) * s4
    c0 = s0 - c1 - c2 - c3 - c4
    hedge = jnp.concatenate(
        [c0[None], c1[None], c2[None], c3[None], c4[None]])

    gn = hn / jnp.sum(hn)
    gnode = hnode / jnp.sum(hnode)
    gedge = hedge / jnp.sum(hedge)
    gval = hval / jnp.sum(hval)

    tn = tn_ref[...]
    tn = tn / jnp.sum(tn)
    tnode = tnode_ref[...]
    tnode = tnode / jnp.sum(tnode)
    tedge = tedge_ref[...]
    tedge = tedge / jnp.sum(tedge)
    tval = tval_ref[...]
    tval = tval / jnp.sum(tval)

    on_ref[...] = gn
    onode_ref[...] = gnode
    oedge_ref[...] = gedge
    oval_ref[...] = gval
    omae_ref[...] = jnp.concatenate([
        jnp.mean(jnp.abs(gn - tn))[None],
        jnp.mean(jnp.abs(gnode - tnode))[None],
        jnp.mean(jnp.abs(gedge - tedge))[None],
        jnp.mean(jnp.abs(gval - tval))[None],
    ])


@jax.jit
def kernel(atom_types, edge_types, n_nodes, n_target_dist, node_target_dist,
           edge_target_dist, valency_target_dist):
    a2 = jnp.asarray(atom_types, jnp.int32).reshape(B * MAX_N)
    e2 = jnp.asarray(edge_types, jnp.int32).reshape(B, MAX_N * MAX_N)
    nn = jnp.asarray(n_nodes, jnp.int32)

    mesh = plsc.VectorSubcoreMesh(core_axis_name="c", subcore_axis_name="s")
    partials = pl.kernel(
        _sc_body,
        out_type=jax.ShapeDtypeStruct((NW, HIST_WORDS), jnp.float32),
        mesh=mesh,
        compiler_params=pltpu.CompilerParams(needs_layout_passes=False),
        scratch_types=[
            pltpu.VMEM((MAX_N * MAX_N,), jnp.int32),
            pltpu.VMEM((MAX_N * MAX_N,), jnp.int32),
            pltpu.VMEM((MAX_N * MAX_N,), jnp.int32),
            pltpu.VMEM((MAX_N * MAX_N,), jnp.int32),
            pltpu.VMEM((MAX_N * MAX_N,), jnp.int32),
            pltpu.VMEM((MAX_N * MAX_N,), jnp.int32),
            pltpu.VMEM((MAX_N * MAX_N,), jnp.int32),
            pltpu.VMEM((MAX_N * MAX_N,), jnp.int32),
            pltpu.VMEM((MPW * MAX_N,), jnp.int32),
            pltpu.VMEM((MPW,), jnp.int32),
            pltpu.VMEM((HIST_WORDS,), jnp.float32),
            pltpu.SemaphoreType.DMA,
            pltpu.SemaphoreType.DMA,
            pltpu.SemaphoreType.DMA,
            pltpu.SemaphoreType.DMA,
            pltpu.SemaphoreType.DMA,
            pltpu.SemaphoreType.DMA,
            pltpu.SemaphoreType.DMA,
            pltpu.SemaphoreType.DMA,
        ],
    )(e2, a2, nn)

    p3 = partials.reshape(NW, HIST_ROWS, L)
    return pl.pallas_call(
        _finalize_body,
        out_shape=(
            jax.ShapeDtypeStruct((MAX_N + 1,), jnp.float32),
            jax.ShapeDtypeStruct((NUM_ATOM_TYPES,), jnp.float32),
            jax.ShapeDtypeStruct((NUM_EDGE_TYPES,), jnp.float32),
            jax.ShapeDtypeStruct((VAL_BINS,), jnp.float32),
            jax.ShapeDtypeStruct((4,), jnp.float32),
        ),
    )(p3, n_target_dist, node_target_dist, edge_target_dist,
      valency_target_dist)


# speedup vs baseline: 1.9786x; 1.1709x over previous
import functools

import jax
import jax.numpy as jnp
from jax import lax
from jax.experimental import pallas as pl
from jax.experimental.pallas import tpu as pltpu
from jax.experimental.pallas import tpu_sc as plsc

B = 8192
MAX_N = 64
NUM_ATOM_TYPES = 16
NUM_EDGE_TYPES = 5
VAL_BINS = 3 * MAX_N - 2

NC = 2
NS = 16
L = 16
NW = NC * NS
MPW = B // NW
GRP = 8

ROW_N = 0
ROW_NODE = ROW_N + MAX_N + 1
ROW_EDGE = ROW_NODE + NUM_ATOM_TYPES
ROW_VAL = ROW_EDGE + NUM_EDGE_TYPES
HIST_ROWS = 288
HIST_WORDS = HIST_ROWS * L


def _sc_body(e_hbm, a_hbm, n_hbm, out_hbm, ebuf0, ebuf1, ebuf2, ebuf3,
             ebuf4, ebuf5, ebuf6, ebuf7, abuf, nbuf, hist, wtab,
             sem0, sem1, sem2, sem3, sem4, sem5, sem6, sem7):
    cid = lax.axis_index("c")
    sid = lax.axis_index("s")
    wid = sid * NC + cid
    base = wid * MPW

    zeros = jnp.zeros((L,), jnp.float32)
    ones = jnp.ones((L,), jnp.float32)
    lane = lax.iota(jnp.int32, L)

    def zero_row(k, _):
        hist[pl.ds(k * L, L)] = zeros
        return 0
    lax.fori_loop(0, HIST_ROWS, zero_row, 0)

    pltpu.sync_copy(n_hbm.at[pl.ds(base, MPW)], nbuf)
    pltpu.sync_copy(a_hbm.at[pl.ds(base * MAX_N, MPW * MAX_N)], abuf)

    sh = jnp.clip((lane - 1) * 8, 0, 24)
    wtab[...] = jnp.where(lane == 0, jnp.int32(0), jnp.int32(1) << sh)

    true_mask = lane < L

    def n_hist(k, _):
        nv = nbuf[pl.ds(k * L, L)]
        idx = (ROW_N + nv) * L + lane
        plsc.addupdate_scatter(hist, [idx], ones, mask=true_mask)
        return 0
    lax.fori_loop(0, MPW // L, n_hist, 0)

    jvecs = [lane + c * L for c in range(MAX_N // L)]
    ebufs = (ebuf0, ebuf1, ebuf2, ebuf3, ebuf4, ebuf5, ebuf6, ebuf7)
    sems = (sem0, sem1, sem2, sem3, sem4, sem5, sem6, sem7)
    NBUF = 8

    def dma_start(m, k):
        src = e_hbm.at[jnp.minimum(base + m, B - 1)]
        pltpu.async_copy(src, ebufs[k], sems[k])

    def dma_wait(k):
        pltpu.make_async_copy(e_hbm.at[base], ebufs[k], sems[k]).wait()

    zero4 = tuple(jnp.zeros((L,), jnp.float32) for _ in range(4))

    izero = jnp.zeros((L,), jnp.int32)

    def process(m, ebuf, cnts):
        nv = plsc.load_gather(nbuf, [jnp.full((L,), m, jnp.int32)])
        n_s = jnp.max(nv)
        colmask = [jv < nv for jv in jvecs]

        def row(i, carry):
            accs, packed = carry
            iv = jnp.full((L,), i, jnp.int32)
            new_accs = []
            for c in range(MAX_N // L):
                et = ebuf[pl.ds(i * MAX_N + c * L, L)]
                etf = et.astype(jnp.float32)
                vf = jnp.where(et == 4, jnp.float32(1.5), etf)
                new_accs.append(accs[c] + vf)
                m1 = colmask[c] & (jvecs[c] > iv)
                w = plsc.load_gather(wtab, [et])
                packed = packed + jnp.where(m1, w, izero)
            return tuple(new_accs), packed

        accs, packed = lax.fori_loop(0, n_s, row, (zero4, izero))

        c1, c2, c3, c4 = cnts
        c1 = c1 + (packed & 0xFF).astype(jnp.float32)
        c2 = c2 + ((packed >> 8) & 0xFF).astype(jnp.float32)
        c3 = c3 + ((packed >> 16) & 0xFF).astype(jnp.float32)
        c4 = c4 + ((packed >> 24) & 0xFF).astype(jnp.float32)
        cnts = (c1, c2, c3, c4)

        for c in range(MAX_N // L):
            vb = jnp.clip(accs[c].astype(jnp.int32), 0, VAL_BINS - 1)
            vidx = (ROW_VAL + vb) * L + lane
            plsc.addupdate_scatter(hist, [vidx], ones, mask=colmask[c])
            at = abuf[pl.ds(m * MAX_N + c * L, L)]
            aidx = (ROW_NODE + at) * L + lane
            plsc.addupdate_scatter(hist, [aidx], ones, mask=colmask[c])
        return cnts

    for k in range(NBUF - 1):
        dma_start(k, k)

    def quad(p, cnts):
        m = p * NBUF
        for k in range(NBUF):
            dma_wait(k)
            dma_start(m + k + NBUF - 1, (k + NBUF - 1) % NBUF)
            cnts = process(m + k, ebufs[k], cnts)
        return cnts

    cnts = lax.fori_loop(0, MPW // NBUF, quad, zero4)
    for k in range(NBUF - 1):
        dma_wait(k)
    for k in range(4):
        hist[pl.ds((ROW_EDGE + 1 + k) * L, L)] = cnts[k]
    pltpu.sync_copy(hist, out_hbm.at[wid])


def _finalize_body(p_ref, tn_ref, tnode_ref, tedge_ref, tval_ref,
                   on_ref, onode_ref, oedge_ref, oval_ref, omae_ref):
    p = p_ref[...]
    s = jnp.sum(jnp.sum(p, axis=0), axis=1)

    hn = s[ROW_N:ROW_N + MAX_N + 1]
    hnode = s[ROW_NODE:ROW_NODE + NUM_ATOM_TYPES]
    hval = s[ROW_VAL:ROW_VAL + VAL_BINS]

    iv = lax.broadcasted_iota(jnp.int32, (1, MAX_N + 1), 1).astype(jnp.float32)
    s0 = jnp.sum(hn.reshape(1, MAX_N + 1) * iv * (iv - 1.0) * 0.5)
    c1 = s[ROW_EDGE + 1]
    c2 = s[ROW_EDGE + 2]
    c3 = s[ROW_EDGE + 3]
    c4 = s[ROW_EDGE + 4]
    c0 = s0 - c1 - c2 - c3 - c4
    hedge = jnp.concatenate(
        [c0[None], c1[None], c2[None], c3[None], c4[None]])

    gn = hn / jnp.sum(hn)
    gnode = hnode / jnp.sum(hnode)
    gedge = hedge / jnp.sum(hedge)
    gval = hval / jnp.sum(hval)

    tn = tn_ref[...]
    tn = tn / jnp.sum(tn)
    tnode = tnode_ref[...]
    tnode = tnode / jnp.sum(tnode)
    tedge = tedge_ref[...]
    tedge = tedge / jnp.sum(tedge)
    tval = tval_ref[...]
    tval = tval / jnp.sum(tval)

    on_ref[...] = gn
    onode_ref[...] = gnode
    oedge_ref[...] = gedge
    oval_ref[...] = gval
    omae_ref[...] = jnp.concatenate([
        jnp.mean(jnp.abs(gn - tn))[None],
        jnp.mean(jnp.abs(gnode - tnode))[None],
        jnp.mean(jnp.abs(gedge - tedge))[None],
        jnp.mean(jnp.abs(gval - tval))[None],
    ])


@jax.jit
def kernel(atom_types, edge_types, n_nodes, n_target_dist, node_target_dist,
           edge_target_dist, valency_target_dist):
    a2 = jnp.asarray(atom_types, jnp.int32).reshape(B * MAX_N)
    e2 = jnp.asarray(edge_types, jnp.int32).reshape(B, MAX_N * MAX_N)
    nn = jnp.asarray(n_nodes, jnp.int32)

    mesh = plsc.VectorSubcoreMesh(core_axis_name="c", subcore_axis_name="s")
    partials = pl.kernel(
        _sc_body,
        out_type=jax.ShapeDtypeStruct((NW, HIST_WORDS), jnp.float32),
        mesh=mesh,
        compiler_params=pltpu.CompilerParams(needs_layout_passes=False),
        scratch_types=[
            pltpu.VMEM((MAX_N * MAX_N,), jnp.int32),
            pltpu.VMEM((MAX_N * MAX_N,), jnp.int32),
            pltpu.VMEM((MAX_N * MAX_N,), jnp.int32),
            pltpu.VMEM((MAX_N * MAX_N,), jnp.int32),
            pltpu.VMEM((MAX_N * MAX_N,), jnp.int32),
            pltpu.VMEM((MAX_N * MAX_N,), jnp.int32),
            pltpu.VMEM((MAX_N * MAX_N,), jnp.int32),
            pltpu.VMEM((MAX_N * MAX_N,), jnp.int32),
            pltpu.VMEM((MPW * MAX_N,), jnp.int32),
            pltpu.VMEM((MPW,), jnp.int32),
            pltpu.VMEM((HIST_WORDS,), jnp.float32),
            pltpu.VMEM((L,), jnp.int32),
            pltpu.SemaphoreType.DMA,
            pltpu.SemaphoreType.DMA,
            pltpu.SemaphoreType.DMA,
            pltpu.SemaphoreType.DMA,
            pltpu.SemaphoreType.DMA,
            pltpu.SemaphoreType.DMA,
            pltpu.SemaphoreType.DMA,
            pltpu.SemaphoreType.DMA,
        ],
    )(e2, a2, nn)

    p3 = partials.reshape(NW, HIST_ROWS, L)
    return pl.pallas_call(
        _finalize_body,
        out_shape=(
            jax.ShapeDtypeStruct((MAX_N + 1,), jnp.float32),
            jax.ShapeDtypeStruct((NUM_ATOM_TYPES,), jnp.float32),
            jax.ShapeDtypeStruct((NUM_EDGE_TYPES,), jnp.float32),
            jax.ShapeDtypeStruct((VAL_BINS,), jnp.float32),
            jax.ShapeDtypeStruct((4,), jnp.float32),
        ),
    )(p3, n_target_dist, node_target_dist, edge_target_dist,
      valency_target_dist)
